# baseline (speedup 1.0000x reference)
import jax
import jax.numpy as jnp
from jax.experimental import pallas as pl
from jax.experimental.pallas import tpu as pltpu

EPS = 1e-5


def _body(sdays_ref, perm_ref, ustep_ref, first_ref, uday_ref, nuniq_ref,
          x_ref, W1_hbm, b1_ref, W2_hbm, b2_ref, g_ref, be_ref, out_ref,
          W1s, W2s, sems):
    i = pl.program_id(0)
    j = pl.program_id(1)
    p = ustep_ref[i]
    slot = jax.lax.rem(p, 2)
    nslot = 1 - slot

    @pl.when((i == 0) & (j == 0))
    def _prologue():
        d0 = uday_ref[0]
        pltpu.make_async_copy(W1_hbm.at[d0], W1s.at[0], sems.at[0, 0]).start()
        pltpu.make_async_copy(W2_hbm.at[d0], W2s.at[0], sems.at[0, 1]).start()

    is_first = (first_ref[i] == 1) & (j == 0)

    @pl.when(is_first)
    def _wait_current():
        d = uday_ref[p]
        pltpu.make_async_copy(W1_hbm.at[d], W1s.at[slot], sems.at[slot, 0]).wait()
        pltpu.make_async_copy(W2_hbm.at[d], W2s.at[slot], sems.at[slot, 1]).wait()

    @pl.when(is_first & (p + 1 < nuniq_ref[0]))
    def _prefetch_next():
        dn = uday_ref[p + 1]
        pltpu.make_async_copy(W1_hbm.at[dn], W1s.at[nslot], sems.at[nslot, 0]).start()
        pltpu.make_async_copy(W2_hbm.at[dn], W2s.at[nslot], sems.at[nslot, 1]).start()

    xb = x_ref[0].astype(jnp.bfloat16)
    h = jnp.dot(xb, W1s[slot].astype(jnp.bfloat16),
                preferred_element_type=jnp.float32)
    h = jnp.maximum(h + b1_ref[0], 0.0).astype(jnp.bfloat16)
    y = jnp.dot(h, W2s[slot].astype(jnp.bfloat16),
                preferred_element_type=jnp.float32)
    y = y + b2_ref[0]
    mu = jnp.mean(y, axis=-1, keepdims=True)
    yc = y - mu
    var = jnp.mean(yc * yc, axis=-1, keepdims=True)
    out_ref[0] = yc * jax.lax.rsqrt(var + EPS) * g_ref[0] + be_ref[0]


def kernel(x, day_indicies, W1, b1, W2, b2, gamma, beta):
    B, T, IN = x.shape
    D, _, HID = W1.shape
    OUT = W2.shape[2]

    day = day_indicies.astype(jnp.int32)
    perm = jnp.argsort(day).astype(jnp.int32)
    sdays = jnp.take(day, perm)

    first = jnp.concatenate(
        [jnp.ones((1,), jnp.int32),
         (sdays[1:] != sdays[:-1]).astype(jnp.int32)])
    ustep = jnp.cumsum(first) - 1
    uday = jnp.zeros((B,), jnp.int32).at[ustep].set(sdays)
    nuniq = jnp.sum(first).reshape(1)

    b1r = b1.reshape(D, 1, HID)
    b2r = b2.reshape(D, 1, OUT)
    gr = gamma.reshape(D, 1, OUT)
    br = beta.reshape(D, 1, OUT)

    grid_spec = pltpu.PrefetchScalarGridSpec(
        num_scalar_prefetch=6,
        grid=(B, 2),
        in_specs=[
            pl.BlockSpec((1, T // 2, IN), lambda i, j, *s: (s[1][i], j, 0)),
            pl.BlockSpec(memory_space=pltpu.MemorySpace.HBM),
            pl.BlockSpec((1, 1, HID), lambda i, j, *s: (s[0][i], 0, 0)),
            pl.BlockSpec(memory_space=pltpu.MemorySpace.HBM),
            pl.BlockSpec((1, 1, OUT), lambda i, j, *s: (s[0][i], 0, 0)),
            pl.BlockSpec((1, 1, OUT), lambda i, j, *s: (s[0][i], 0, 0)),
            pl.BlockSpec((1, 1, OUT), lambda i, j, *s: (s[0][i], 0, 0)),
        ],
        out_specs=pl.BlockSpec((1, T // 2, OUT),
                               lambda i, j, *s: (s[1][i], j, 0)),
        scratch_shapes=[
            pltpu.VMEM((2, IN, HID), jnp.float32),
            pltpu.VMEM((2, HID, OUT), jnp.float32),
            pltpu.SemaphoreType.DMA((2, 2)),
        ],
    )

    return pl.pallas_call(
        _body,
        grid_spec=grid_spec,
        out_shape=jax.ShapeDtypeStruct((B, T, OUT), jnp.float32),
        compiler_params=pltpu.CompilerParams(
            dimension_semantics=("arbitrary", "arbitrary"),
        ),
    )(sdays, perm, ustep, first, uday, nuniq,
      x, W1, b1r, W2, b2r, gr, br)


# speedup vs baseline: 1.3861x; 1.3861x over previous
import jax
import jax.numpy as jnp
from jax import lax
from jax.experimental import pallas as pl
from jax.experimental.pallas import tpu as pltpu

EPS = 1e-5


def _body(perm_ref, ustep_ref, first_ref, uday_ref, nuniq_ref,
          x_hbm, W1_hbm, b1_ref, W2_hbm, b2_ref, g_ref, be_ref, out_hbm,
          Xs, Ys, W1s, W2s, xsem, ysem, wsem):
    B = x_hbm.shape[0]
    nu = nuniq_ref[0]

    pltpu.make_async_copy(x_hbm.at[perm_ref[0]], Xs.at[0], xsem.at[0]).start()
    pltpu.make_async_copy(x_hbm.at[perm_ref[1]], Xs.at[1], xsem.at[1]).start()
    d0 = uday_ref[0]
    pltpu.make_async_copy(W1_hbm.at[d0], W1s.at[0], wsem.at[0, 0]).start()
    pltpu.make_async_copy(W2_hbm.at[d0], W2s.at[0], wsem.at[0, 1]).start()

    def step(s, carry):
        p = ustep_ref[s]
        slot = lax.rem(p, 2)
        xslot = lax.rem(s, 3)

        @pl.when(s + 2 < B)
        def _():
            pltpu.make_async_copy(x_hbm.at[perm_ref[s + 2]],
                                  Xs.at[lax.rem(s + 2, 3)],
                                  xsem.at[lax.rem(s + 2, 3)]).start()

        is_first = first_ref[s] == 1

        @pl.when(is_first)
        def _():
            d = uday_ref[p]
            pltpu.make_async_copy(W1_hbm.at[d], W1s.at[slot],
                                  wsem.at[slot, 0]).wait()
            pltpu.make_async_copy(W2_hbm.at[d], W2s.at[slot],
                                  wsem.at[slot, 1]).wait()

        @pl.when(is_first & (p + 1 < nu))
        def _():
            dn = uday_ref[p + 1]
            nslot = 1 - slot
            pltpu.make_async_copy(W1_hbm.at[dn], W1s.at[nslot],
                                  wsem.at[nslot, 0]).start()
            pltpu.make_async_copy(W2_hbm.at[dn], W2s.at[nslot],
                                  wsem.at[nslot, 1]).start()

        pltpu.make_async_copy(x_hbm.at[perm_ref[s]], Xs.at[xslot],
                              xsem.at[xslot]).wait()

        @pl.when(s >= 3)
        def _():
            pltpu.make_async_copy(Ys.at[xslot], out_hbm.at[perm_ref[s - 3]],
                                  ysem.at[xslot]).wait()

        d = uday_ref[p]
        xb = Xs[xslot].astype(jnp.bfloat16)
        h = jnp.dot(xb, W1s[slot].astype(jnp.bfloat16),
                    preferred_element_type=jnp.float32)
        h = jnp.maximum(h + b1_ref[d], 0.0).astype(jnp.bfloat16)
        y = jnp.dot(h, W2s[slot].astype(jnp.bfloat16),
                    preferred_element_type=jnp.float32)
        y = y + b2_ref[d]
        mu = jnp.mean(y, axis=-1, keepdims=True)
        yc = y - mu
        var = jnp.mean(yc * yc, axis=-1, keepdims=True)
        Ys[xslot] = yc * lax.rsqrt(var + EPS) * g_ref[d] + be_ref[d]

        pltpu.make_async_copy(Ys.at[xslot], out_hbm.at[perm_ref[s]],
                              ysem.at[xslot]).start()
        return carry

    lax.fori_loop(0, B, step, 0, unroll=False)

    for k in range(3):
        s = B - 3 + k
        pltpu.make_async_copy(Ys.at[lax.rem(s, 3)],
                              out_hbm.at[perm_ref[s]],
                              ysem.at[lax.rem(s, 3)]).wait()


def kernel(x, day_indicies, W1, b1, W2, b2, gamma, beta):
    B, T, IN = x.shape
    D, _, HID = W1.shape
    OUT = W2.shape[2]

    day = day_indicies.astype(jnp.int32)
    perm = jnp.argsort(day).astype(jnp.int32)
    sdays = jnp.take(day, perm)

    first = jnp.concatenate(
        [jnp.ones((1,), jnp.int32),
         (sdays[1:] != sdays[:-1]).astype(jnp.int32)])
    ustep = jnp.cumsum(first) - 1
    uday = jnp.zeros((B,), jnp.int32).at[ustep].set(sdays)
    nuniq = jnp.sum(first).reshape(1)

    b1r = b1.reshape(D, 1, HID)
    b2r = b2.reshape(D, 1, OUT)
    gr = gamma.reshape(D, 1, OUT)
    br = beta.reshape(D, 1, OUT)

    vec_spec = pl.BlockSpec(memory_space=pltpu.MemorySpace.VMEM)
    hbm = pl.BlockSpec(memory_space=pltpu.MemorySpace.HBM)

    grid_spec = pltpu.PrefetchScalarGridSpec(
        num_scalar_prefetch=5,
        grid=(1,),
        in_specs=[hbm, hbm, vec_spec, hbm, vec_spec, vec_spec, vec_spec],
        out_specs=hbm,
        scratch_shapes=[
            pltpu.VMEM((3, T, IN), jnp.float32),
            pltpu.VMEM((3, T, OUT), jnp.float32),
            pltpu.VMEM((2, IN, HID), jnp.float32),
            pltpu.VMEM((2, HID, OUT), jnp.float32),
            pltpu.SemaphoreType.DMA((3,)),
            pltpu.SemaphoreType.DMA((3,)),
            pltpu.SemaphoreType.DMA((2, 2)),
        ],
    )

    return pl.pallas_call(
        _body,
        grid_spec=grid_spec,
        out_shape=jax.ShapeDtypeStruct((B, T, OUT), jnp.float32),
        compiler_params=pltpu.CompilerParams(
            dimension_semantics=("arbitrary",),
        ),
    )(perm, ustep, first, uday, nuniq,
      x, W1, b1r, W2, b2r, gr, br)
